# baseline (speedup 1.0000x reference)
import functools

import jax
import jax.numpy as jnp
from jax import lax
from jax.experimental import pallas as pl
from jax.experimental.pallas import tpu as pltpu
from jax.experimental.pallas import tpu_sc as plsc

N = 10000
E = 320000
D = 128
NPAD = 10240
NC = 2
NS = 16
NW = NC * NS
EPT = E // NW
CH = 128
NBUF = 2
GRP = 8
NG = 10
NCHUNK = NG * GRP
EPTP = NCHUNK * CH
NGS = NG + 1
RPT = NPAD // NS

_MESH = plsc.VectorSubcoreMesh(
    core_axis_name="c", subcore_axis_name="s", num_cores=NC, num_subcores=NS)


def _fill(buf, rows, val):
  v = jnp.full((16,), val, jnp.float32)

  def body(i, _):
    for k in range(D // 16):
      buf[i, pl.ds(k * 16, 16)] = v
    return 0

  lax.fori_loop(0, rows, body, 0)


def _zero_acc(acc, rows_buf, sid):
  _fill(rows_buf, CH, 0.0)
  base = sid * RPT
  nfull = RPT // CH
  for k in range(nfull):
    pltpu.sync_copy(rows_buf, acc.at[pl.ds(base + k * CH, CH), :])
  rem = RPT - nfull * CH
  if rem:
    pltpu.sync_copy(rows_buf.at[pl.ds(0, rem), :],
                    acc.at[pl.ds(base + nfull * CH, rem), :])


def _writeout(acc, out_hbm, cid, sid):
  pltpu.sync_copy(acc.at[pl.ds(sid * RPT, RPT), :],
                  out_hbm.at[cid, pl.ds(sid * RPT, RPT), :])


@functools.partial(
    pl.kernel,
    out_type=jax.ShapeDtypeStruct((NC, NPAD, D), jnp.float32),
    mesh=_MESH,
    scratch_types=[
        pltpu.VMEM((GRP, CH), jnp.int32),
        pltpu.VMEM((CH, D), jnp.float32),
        pltpu.MemorySpace.VMEM_SHARED((NPAD, D), jnp.float32),
        pltpu.SemaphoreType.DMA,
    ],
)
def _deg_kernel(dst4_hbm, out_hbm, dbi, rows_buf, acc, sem):
  cid = lax.axis_index("c")
  sid = lax.axis_index("s")
  wid = cid * NS + sid
  _zero_acc(acc, rows_buf, sid)
  plsc.subcore_barrier()
  _fill(rows_buf, CH, 1.0)

  def group(g, _):
    pltpu.sync_copy(dst4_hbm.at[wid, g], dbi)
    for b in range(GRP):
      pltpu.async_copy(rows_buf, acc.at[dbi.at[b]], sem, add=True)
    for b in range(GRP):
      pltpu.make_async_copy(rows_buf, acc.at[dbi.at[b]], sem).wait()
    return 0

  lax.fori_loop(0, NG, group, 0)
  plsc.subcore_barrier()
  _writeout(acc, out_hbm, cid, sid)


@functools.partial(
    pl.kernel,
    out_type=jax.ShapeDtypeStruct((NC, NPAD, D), jnp.float32),
    mesh=_MESH,
    scratch_types=[
        pltpu.VMEM((2, GRP, CH), jnp.int32),
        pltpu.VMEM((GRP, CH), jnp.int32),
        [pltpu.VMEM((CH, D), jnp.float32)] * NBUF,
        pltpu.MemorySpace.VMEM_SHARED((NPAD, D), jnp.float32),
        [pltpu.SemaphoreType.DMA] * NBUF,
    ],
)
def _agg_kernel(h_hbm, src4_hbm, dst4_hbm, out_hbm, sbi, dbi, ring, acc, sems):
  cid = lax.axis_index("c")
  sid = lax.axis_index("s")
  wid = cid * NS + sid
  _zero_acc(acc, ring[0], sid)
  pltpu.sync_copy(src4_hbm.at[wid, 0], sbi.at[0])
  plsc.subcore_barrier()

  for b in range(NBUF):
    pltpu.async_copy(h_hbm.at[sbi.at[0, b]], ring[b], sems[b])

  def group(g, _):
    pltpu.sync_copy(src4_hbm.at[wid, g + 1], sbi.at[(g + 1) % 2])
    pltpu.sync_copy(dst4_hbm.at[wid, g], dbi)
    for b in range(GRP):
      s = b % NBUF
      pltpu.make_async_copy(h_hbm.at[sbi.at[g % 2, b]], ring[s],
                            sems[s]).wait()
      pltpu.sync_copy(ring[s], acc.at[dbi.at[b]], add=True)
      if b + NBUF < GRP:
        pltpu.async_copy(h_hbm.at[sbi.at[g % 2, b + NBUF]], ring[s], sems[s])
      else:
        pltpu.async_copy(h_hbm.at[sbi.at[(g + 1) % 2, b + NBUF - GRP]],
                         ring[s], sems[s])
    return 0

  lax.fori_loop(0, NG, group, 0)
  for b in range(NBUF):
    pltpu.make_async_copy(h_hbm.at[sbi.at[0, b]], ring[b], sems[b]).wait()

  plsc.subcore_barrier()
  _writeout(acc, out_hbm, cid, sid)


BN = 2048


def _self_body(h_ref, ws_ref, b_ref, o_ref):
  o_ref[...] = jnp.dot(h_ref[...], ws_ref[...],
                       preferred_element_type=jnp.float32) + b_ref[...]


def _self_matmul(h, ws, b):
  return pl.pallas_call(
      _self_body,
      grid=(NPAD // BN,),
      in_specs=[
          pl.BlockSpec((BN, D), lambda i: (i, 0)),
          pl.BlockSpec((D, D), lambda i: (0, 0)),
          pl.BlockSpec((1, D), lambda i: (0, 0)),
      ],
      out_specs=pl.BlockSpec((BN, D), lambda i: (i, 0)),
      out_shape=jax.ShapeDtypeStruct((NPAD, D), jnp.float32),
  )(h, ws, b)


def _combine_body(act, s_ref, p_ref, dg_ref, wn_ref, o_ref):
  deg = jnp.maximum(dg_ref[0] + dg_ref[1], 1.0)
  neigh = (p_ref[0] + p_ref[1]) / deg
  out = s_ref[...] + jnp.dot(neigh, wn_ref[...],
                             preferred_element_type=jnp.float32)
  if act:
    out = jnp.maximum(out, 0.0)
  o_ref[...] = out


def _combine(s, p, dg, wn, act):
  grid = (NPAD // BN,)
  return pl.pallas_call(
      functools.partial(_combine_body, act),
      grid=grid,
      in_specs=[
          pl.BlockSpec((BN, D), lambda i: (i, 0)),
          pl.BlockSpec((NC, BN, D), lambda i: (0, i, 0)),
          pl.BlockSpec((NC, BN, D), lambda i: (0, i, 0)),
          pl.BlockSpec((D, D), lambda i: (0, 0)),
      ],
      out_specs=pl.BlockSpec((BN, D), lambda i: (i, 0)),
      out_shape=jax.ShapeDtypeStruct((NPAD, D), jnp.float32),
  )(s, p, dg, wn)


def kernel(features, edge_index, Ws0, Wn0, b0, Ws1, Wn1, b1, Ws2, Wn2, b2):
  src = edge_index[0]
  dst = edge_index[1]
  nps = NGS * GRP * CH - EPT
  npd = EPTP - EPT
  pad_src = jnp.broadcast_to((jnp.arange(nps, dtype=jnp.int32) * 89) % N,
                             (NW, nps))
  pad_dst = jnp.broadcast_to(
      N + (jnp.arange(npd, dtype=jnp.int32) % (NPAD - N)), (NW, npd))
  src4 = jnp.concatenate([src.reshape(NW, EPT), pad_src], axis=1)
  src4 = src4.reshape(NW, NGS, GRP, CH)
  dst4 = jnp.concatenate([dst.reshape(NW, EPT), pad_dst], axis=1)
  dst4 = dst4.reshape(NW, NG, GRP, CH)

  h = jnp.zeros((NPAD, D), jnp.float32).at[:N].set(features)
  dg = _deg_kernel(dst4)

  layers = ((Ws0, Wn0, b0, True), (Ws1, Wn1, b1, True), (Ws2, Wn2, b2, False))
  for ws, wn, b, act in layers:
    p = _agg_kernel(h, src4, dst4)
    s = _self_matmul(h, ws, b.reshape(1, D))
    h = _combine(s, p, dg, wn, act)
  return h[:N]


# speedup vs baseline: 1.1429x; 1.1429x over previous
import functools

import jax
import jax.numpy as jnp
from jax import lax
from jax.experimental import pallas as pl
from jax.experimental.pallas import tpu as pltpu
from jax.experimental.pallas import tpu_sc as plsc

N = 10000
E = 320000
D = 128
NPAD = 10240
NC = 2
NS = 16
NW = NC * NS
EPT = E // NW
CH = 128
NBUF = 2
GRP = 8
NG = 10
NCHUNK = NG * GRP
EPTP = NCHUNK * CH
NGS = NG + 1
RPT = NPAD // NS

_MESH = plsc.VectorSubcoreMesh(
    core_axis_name="c", subcore_axis_name="s", num_cores=NC, num_subcores=NS)


def _fill(buf, rows, val):
  v = jnp.full((16,), val, jnp.float32)

  def body(i, _):
    for k in range(D // 16):
      buf[i, pl.ds(k * 16, 16)] = v
    return 0

  lax.fori_loop(0, rows, body, 0)


def _zero_acc(acc, rows_buf, sid):
  _fill(rows_buf, CH, 0.0)
  base = sid * RPT
  nfull = RPT // CH
  for k in range(nfull):
    pltpu.sync_copy(rows_buf, acc.at[pl.ds(base + k * CH, CH), :])
  rem = RPT - nfull * CH
  if rem:
    pltpu.sync_copy(rows_buf.at[pl.ds(0, rem), :],
                    acc.at[pl.ds(base + nfull * CH, rem), :])


def _writeout(acc, out_hbm, cid, sid):
  pltpu.sync_copy(acc.at[pl.ds(sid * RPT, RPT), :],
                  out_hbm.at[cid, pl.ds(sid * RPT, RPT), :])


def _fill1d(buf, n, val):
  v = jnp.full((16,), val, jnp.float32)
  for k in range(n // 16):
    buf[pl.ds(k * 16, 16)] = v


@functools.partial(
    pl.kernel,
    out_type=jax.ShapeDtypeStruct((NC * NPAD,), jnp.float32),
    mesh=_MESH,
    scratch_types=[
        pltpu.VMEM((GRP, CH), jnp.int32),
        pltpu.VMEM((CH,), jnp.float32),
        pltpu.VMEM((RPT,), jnp.float32),
        pltpu.MemorySpace.VMEM_SHARED((NPAD,), jnp.float32),
        pltpu.SemaphoreType.DMA,
    ],
)
def _deg_kernel(dst4_hbm, out_hbm, dbi, ones, zbuf, acc, sem):
  cid = lax.axis_index("c")
  sid = lax.axis_index("s")
  wid = cid * NS + sid
  _fill1d(zbuf, RPT, 0.0)
  pltpu.sync_copy(zbuf, acc.at[pl.ds(sid * RPT, RPT)])
  _fill1d(ones, CH, 1.0)
  plsc.subcore_barrier()

  def group(g, _):
    pltpu.sync_copy(dst4_hbm.at[wid, g], dbi)
    for b in range(GRP):
      pltpu.async_copy(ones, acc.at[dbi.at[b]], sem, add=True)
    for b in range(GRP):
      pltpu.make_async_copy(ones, acc.at[dbi.at[b]], sem).wait()
    return 0

  lax.fori_loop(0, NG, group, 0)
  plsc.subcore_barrier()
  pltpu.sync_copy(acc.at[pl.ds(sid * RPT, RPT)],
                  out_hbm.at[pl.ds(cid * NPAD + sid * RPT, RPT)])


NPT = NPAD // NW


@functools.partial(
    pl.kernel,
    out_type=jax.ShapeDtypeStruct((NPAD, D), jnp.float32),
    mesh=_MESH,
    scratch_types=[
        pltpu.VMEM((NPT,), jnp.float32),
        pltpu.VMEM((NPT,), jnp.float32),
        pltpu.VMEM((NPT, D), jnp.float32),
    ],
)
def _invdeg_kernel(dgp_hbm, out_hbm, d0, d1, rows):
  cid = lax.axis_index("c")
  sid = lax.axis_index("s")
  wid = cid * NS + sid
  pltpu.sync_copy(dgp_hbm.at[pl.ds(wid * NPT, NPT)], d0)
  pltpu.sync_copy(dgp_hbm.at[pl.ds(NPAD + wid * NPT, NPT)], d1)
  for k in range(NPT // 16):
    sl = pl.ds(k * 16, 16)
    d0[sl] = 1.0 / jnp.maximum(d0[sl] + d1[sl], 1.0)

  def row16(m, _):
    w = d0[pl.ds(m * 16, 16)]
    for i in range(16):
      v = jnp.full((16,), w[i], jnp.float32)
      for k in range(D // 16):
        rows[m * 16 + i, pl.ds(k * 16, 16)] = v
    return 0

  lax.fori_loop(0, NPT // 16, row16, 0)
  pltpu.sync_copy(rows, out_hbm.at[pl.ds(wid * NPT, NPT), :])


@functools.partial(
    pl.kernel,
    out_type=jax.ShapeDtypeStruct((NC, NPAD, D), jnp.float32),
    mesh=_MESH,
    scratch_types=[
        pltpu.VMEM((2, GRP, CH), jnp.int32),
        pltpu.VMEM((GRP, CH), jnp.int32),
        [pltpu.VMEM((CH, D), jnp.float32)] * NBUF,
        pltpu.MemorySpace.VMEM_SHARED((NPAD, D), jnp.float32),
        [pltpu.SemaphoreType.DMA] * NBUF,
    ],
)
def _agg_kernel(h_hbm, src4_hbm, dst4_hbm, out_hbm, sbi, dbi, ring, acc, sems):
  cid = lax.axis_index("c")
  sid = lax.axis_index("s")
  wid = cid * NS + sid
  _zero_acc(acc, ring[0], sid)
  pltpu.sync_copy(src4_hbm.at[wid, 0], sbi.at[0])
  plsc.subcore_barrier()

  for b in range(NBUF):
    pltpu.async_copy(h_hbm.at[sbi.at[0, b]], ring[b], sems[b])

  def group(g, _):
    pltpu.sync_copy(src4_hbm.at[wid, g + 1], sbi.at[(g + 1) % 2])
    pltpu.sync_copy(dst4_hbm.at[wid, g], dbi)
    for b in range(GRP):
      s = b % NBUF
      pltpu.make_async_copy(h_hbm.at[sbi.at[g % 2, b]], ring[s],
                            sems[s]).wait()
      pltpu.sync_copy(ring[s], acc.at[dbi.at[b]], add=True)
      if b + NBUF < GRP:
        pltpu.async_copy(h_hbm.at[sbi.at[g % 2, b + NBUF]], ring[s], sems[s])
      else:
        pltpu.async_copy(h_hbm.at[sbi.at[(g + 1) % 2, b + NBUF - GRP]],
                         ring[s], sems[s])
    return 0

  lax.fori_loop(0, NG, group, 0)
  for b in range(NBUF):
    pltpu.make_async_copy(h_hbm.at[sbi.at[0, b]], ring[b], sems[b]).wait()

  plsc.subcore_barrier()
  _writeout(acc, out_hbm, cid, sid)


BN = 2048


def _combine_body(act, h_ref, p_ref, iv_ref, ws_ref, wn_ref, b_ref, o_ref):
  neigh = (p_ref[0] + p_ref[1]) * iv_ref[...]
  out = (jnp.dot(h_ref[...], ws_ref[...], preferred_element_type=jnp.float32)
         + jnp.dot(neigh, wn_ref[...], preferred_element_type=jnp.float32)
         + b_ref[...])
  if act:
    out = jnp.maximum(out, 0.0)
  o_ref[...] = out


def _combine(h, p, iv, ws, wn, b, act):
  grid = (NPAD // BN,)
  return pl.pallas_call(
      functools.partial(_combine_body, act),
      grid=grid,
      in_specs=[
          pl.BlockSpec((BN, D), lambda i: (i, 0)),
          pl.BlockSpec((NC, BN, D), lambda i: (0, i, 0)),
          pl.BlockSpec((BN, D), lambda i: (i, 0)),
          pl.BlockSpec((D, D), lambda i: (0, 0)),
          pl.BlockSpec((D, D), lambda i: (0, 0)),
          pl.BlockSpec((1, D), lambda i: (0, 0)),
      ],
      out_specs=pl.BlockSpec((BN, D), lambda i: (i, 0)),
      out_shape=jax.ShapeDtypeStruct((NPAD, D), jnp.float32),
  )(h, p, iv, ws, wn, b)


def kernel(features, edge_index, Ws0, Wn0, b0, Ws1, Wn1, b1, Ws2, Wn2, b2):
  src = edge_index[0]
  dst = edge_index[1]
  nps = NGS * GRP * CH - EPT
  npd = EPTP - EPT
  pad_src = jnp.broadcast_to((jnp.arange(nps, dtype=jnp.int32) * 89) % N,
                             (NW, nps))
  pad_dst = jnp.broadcast_to(
      N + (jnp.arange(npd, dtype=jnp.int32) % (NPAD - N)), (NW, npd))
  src4 = jnp.concatenate([src.reshape(NW, EPT), pad_src], axis=1)
  src4 = src4.reshape(NW, NGS, GRP, CH)
  dst4 = jnp.concatenate([dst.reshape(NW, EPT), pad_dst], axis=1)
  dst4 = dst4.reshape(NW, NG, GRP, CH)

  h = jnp.zeros((NPAD, D), jnp.float32).at[:N].set(features)
  iv = _invdeg_kernel(_deg_kernel(dst4))

  layers = ((Ws0, Wn0, b0, True), (Ws1, Wn1, b1, True), (Ws2, Wn2, b2, False))
  for ws, wn, b, act in layers:
    p = _agg_kernel(h, src4, dst4)
    h = _combine(h, p, iv, ws, wn, b.reshape(1, D), act)
  return h[:N]


# speedup vs baseline: 1.1667x; 1.0208x over previous
import functools

import jax
import jax.numpy as jnp
from jax import lax
from jax.experimental import pallas as pl
from jax.experimental.pallas import tpu as pltpu
from jax.experimental.pallas import tpu_sc as plsc

N = 10000
E = 320000
D = 128
NPAD = 10240
NC = 2
NS = 16
NW = NC * NS
EPT = E // NW
CH = 128
NBUF = 2
GRP = 16
NG = 5
NCHUNK = NG * GRP
EPTP = NCHUNK * CH
NGS = NG + 1
RPT = NPAD // NS

_MESH = plsc.VectorSubcoreMesh(
    core_axis_name="c", subcore_axis_name="s", num_cores=NC, num_subcores=NS)


def _fill(buf, rows, val):
  v = jnp.full((16,), val, jnp.float32)

  def body(i, _):
    for k in range(D // 16):
      buf[i, pl.ds(k * 16, 16)] = v
    return 0

  lax.fori_loop(0, rows, body, 0)


def _zero_acc(acc, rows_buf, sid):
  _fill(rows_buf, CH, 0.0)
  base = sid * RPT
  nfull = RPT // CH
  for k in range(nfull):
    pltpu.sync_copy(rows_buf, acc.at[pl.ds(base + k * CH, CH), :])
  rem = RPT - nfull * CH
  if rem:
    pltpu.sync_copy(rows_buf.at[pl.ds(0, rem), :],
                    acc.at[pl.ds(base + nfull * CH, rem), :])


def _writeout(acc, out_hbm, cid, sid):
  pltpu.sync_copy(acc.at[pl.ds(sid * RPT, RPT), :],
                  out_hbm.at[cid, pl.ds(sid * RPT, RPT), :])


def _fill1d(buf, n, val):
  v = jnp.full((16,), val, jnp.float32)
  for k in range(n // 16):
    buf[pl.ds(k * 16, 16)] = v


@functools.partial(
    pl.kernel,
    out_type=jax.ShapeDtypeStruct((NC * NPAD,), jnp.float32),
    mesh=_MESH,
    scratch_types=[
        pltpu.VMEM((GRP, CH), jnp.int32),
        pltpu.VMEM((CH,), jnp.float32),
        pltpu.VMEM((RPT,), jnp.float32),
        pltpu.MemorySpace.VMEM_SHARED((NPAD,), jnp.float32),
        pltpu.SemaphoreType.DMA,
    ],
)
def _deg_kernel(dst4_hbm, out_hbm, dbi, ones, zbuf, acc, sem):
  cid = lax.axis_index("c")
  sid = lax.axis_index("s")
  wid = cid * NS + sid
  _fill1d(zbuf, RPT, 0.0)
  pltpu.sync_copy(zbuf, acc.at[pl.ds(sid * RPT, RPT)])
  _fill1d(ones, CH, 1.0)
  plsc.subcore_barrier()

  def group(g, _):
    pltpu.sync_copy(dst4_hbm.at[wid, g], dbi)
    for b in range(GRP):
      pltpu.async_copy(ones, acc.at[dbi.at[b]], sem, add=True)
    for b in range(GRP):
      pltpu.make_async_copy(ones, acc.at[dbi.at[b]], sem).wait()
    return 0

  lax.fori_loop(0, NG, group, 0)
  plsc.subcore_barrier()
  pltpu.sync_copy(acc.at[pl.ds(sid * RPT, RPT)],
                  out_hbm.at[pl.ds(cid * NPAD + sid * RPT, RPT)])


NPT = NPAD // NW


@functools.partial(
    pl.kernel,
    out_type=jax.ShapeDtypeStruct((NPAD, D), jnp.float32),
    mesh=_MESH,
    scratch_types=[
        pltpu.VMEM((NPT,), jnp.float32),
        pltpu.VMEM((NPT,), jnp.float32),
        pltpu.VMEM((NPT, D), jnp.float32),
    ],
)
def _invdeg_kernel(dgp_hbm, out_hbm, d0, d1, rows):
  cid = lax.axis_index("c")
  sid = lax.axis_index("s")
  wid = cid * NS + sid
  pltpu.sync_copy(dgp_hbm.at[pl.ds(wid * NPT, NPT)], d0)
  pltpu.sync_copy(dgp_hbm.at[pl.ds(NPAD + wid * NPT, NPT)], d1)
  for k in range(NPT // 16):
    sl = pl.ds(k * 16, 16)
    d0[sl] = 1.0 / jnp.maximum(d0[sl] + d1[sl], 1.0)

  def row16(m, _):
    w = d0[pl.ds(m * 16, 16)]
    for i in range(16):
      v = jnp.full((16,), w[i], jnp.float32)
      for k in range(D // 16):
        rows[m * 16 + i, pl.ds(k * 16, 16)] = v
    return 0

  lax.fori_loop(0, NPT // 16, row16, 0)
  pltpu.sync_copy(rows, out_hbm.at[pl.ds(wid * NPT, NPT), :])


@functools.partial(
    pl.kernel,
    out_type=jax.ShapeDtypeStruct((NC, NPAD, D), jnp.float32),
    mesh=_MESH,
    scratch_types=[
        pltpu.VMEM((2, GRP, CH), jnp.int32),
        pltpu.VMEM((GRP, CH), jnp.int32),
        [pltpu.VMEM((CH, D), jnp.float32)] * NBUF,
        pltpu.MemorySpace.VMEM_SHARED((NPAD, D), jnp.float32),
        [pltpu.SemaphoreType.DMA] * NBUF,
    ],
)
def _agg_kernel(h_hbm, src4_hbm, dst4_hbm, out_hbm, sbi, dbi, ring, acc, sems):
  cid = lax.axis_index("c")
  sid = lax.axis_index("s")
  wid = cid * NS + sid
  _zero_acc(acc, ring[0], sid)
  pltpu.sync_copy(src4_hbm.at[wid, 0], sbi.at[0])
  plsc.subcore_barrier()

  for b in range(NBUF):
    pltpu.async_copy(h_hbm.at[sbi.at[0, b]], ring[b], sems[b])

  def group(g, _):
    pltpu.sync_copy(src4_hbm.at[wid, g + 1], sbi.at[(g + 1) % 2])
    pltpu.sync_copy(dst4_hbm.at[wid, g], dbi)
    for b in range(GRP):
      s = b % NBUF
      pltpu.make_async_copy(h_hbm.at[sbi.at[g % 2, b]], ring[s],
                            sems[s]).wait()
      pltpu.sync_copy(ring[s], acc.at[dbi.at[b]], add=True)
      if b + NBUF < GRP:
        pltpu.async_copy(h_hbm.at[sbi.at[g % 2, b + NBUF]], ring[s], sems[s])
      else:
        pltpu.async_copy(h_hbm.at[sbi.at[(g + 1) % 2, b + NBUF - GRP]],
                         ring[s], sems[s])
    return 0

  lax.fori_loop(0, NG, group, 0)
  for b in range(NBUF):
    pltpu.make_async_copy(h_hbm.at[sbi.at[0, b]], ring[b], sems[b]).wait()

  plsc.subcore_barrier()
  _writeout(acc, out_hbm, cid, sid)


BN = 2048


def _combine_body(act, h_ref, p_ref, iv_ref, ws_ref, wn_ref, b_ref, o_ref):
  neigh = (p_ref[0] + p_ref[1]) * iv_ref[...]
  out = (jnp.dot(h_ref[...], ws_ref[...], preferred_element_type=jnp.float32)
         + jnp.dot(neigh, wn_ref[...], preferred_element_type=jnp.float32)
         + b_ref[...])
  if act:
    out = jnp.maximum(out, 0.0)
  o_ref[...] = out


def _combine(h, p, iv, ws, wn, b, act):
  grid = (NPAD // BN,)
  return pl.pallas_call(
      functools.partial(_combine_body, act),
      grid=grid,
      in_specs=[
          pl.BlockSpec((BN, D), lambda i: (i, 0)),
          pl.BlockSpec((NC, BN, D), lambda i: (0, i, 0)),
          pl.BlockSpec((BN, D), lambda i: (i, 0)),
          pl.BlockSpec((D, D), lambda i: (0, 0)),
          pl.BlockSpec((D, D), lambda i: (0, 0)),
          pl.BlockSpec((1, D), lambda i: (0, 0)),
      ],
      out_specs=pl.BlockSpec((BN, D), lambda i: (i, 0)),
      out_shape=jax.ShapeDtypeStruct((NPAD, D), jnp.float32),
  )(h, p, iv, ws, wn, b)


def kernel(features, edge_index, Ws0, Wn0, b0, Ws1, Wn1, b1, Ws2, Wn2, b2):
  src = edge_index[0]
  dst = edge_index[1]
  nps = NGS * GRP * CH - EPT
  npd = EPTP - EPT
  pad_src = jnp.broadcast_to((jnp.arange(nps, dtype=jnp.int32) * 89) % N,
                             (NW, nps))
  pad_dst = jnp.broadcast_to(
      N + (jnp.arange(npd, dtype=jnp.int32) % (NPAD - N)), (NW, npd))
  src4 = jnp.concatenate([src.reshape(NW, EPT), pad_src], axis=1)
  src4 = src4.reshape(NW, NGS, GRP, CH)
  dst4 = jnp.concatenate([dst.reshape(NW, EPT), pad_dst], axis=1)
  dst4 = dst4.reshape(NW, NG, GRP, CH)

  h = jnp.zeros((NPAD, D), jnp.float32).at[:N].set(features)
  iv = _invdeg_kernel(_deg_kernel(dst4))

  layers = ((Ws0, Wn0, b0, True), (Ws1, Wn1, b1, True), (Ws2, Wn2, b2, False))
  for ws, wn, b, act in layers:
    p = _agg_kernel(h, src4, dst4)
    h = _combine(h, p, iv, ws, wn, b.reshape(1, D), act)
  return h[:N]
